# baseline (speedup 1.0000x reference)
import jax
import jax.numpy as jnp
from jax.experimental import pallas as pl
from jax.experimental.pallas import tpu as pltpu

R = 2
H = 2
N = 4096
DIMF = 128
HID = 64
ALPHA = 0.2

BM = 256


def _prologue_compute(x, xr, wua, wna, wnc, whn_s, a_s, b_s):
    ones = jnp.ones((N, 1), jnp.float32)
    zer = jnp.zeros((N, 63), jnp.float32)
    whn = jnp.dot(xr, wnc, preferred_element_type=jnp.float32)
    whn_s[...] = jnp.concatenate(
        [whn[:, 0:HID], ones, zer, whn[:, HID:2 * HID], ones, zer],
        axis=1).astype(jnp.bfloat16)
    su = jnp.dot(x, wua, preferred_element_type=jnp.float32)
    a_s[...] = jnp.concatenate(
        [jnp.exp((1.0 - ALPHA) * su[:, 0:H]),
         jnp.zeros((N, 8 - H), jnp.float32)], axis=1).astype(jnp.bfloat16)
    sn = jnp.dot(xr, wna, preferred_element_type=jnp.float32)
    b_s[...] = jnp.concatenate(
        [jnp.exp(sn[:, 0:H]), jnp.exp(ALPHA * sn[:, 0:H]),
         jnp.zeros((N, 8 - 2 * H), jnp.float32)],
        axis=1).T.astype(jnp.bfloat16)


def _att_block(adj_refs, whn_s, a_s, b_s, i):
    nh = len(adj_refs)
    cw = N // nh
    a = a_s[pl.ds(i * BM, BM), :]
    nds = []
    for h in range(H):
        nd = jnp.zeros((BM, 128), jnp.float32)
        for j, adj_ref in enumerate(adj_refs):
            adjw = adj_ref[...].astype(jnp.bfloat16)
            q1 = a[:, h:h + 1] * b_s[h:h + 1, pl.ds(j * cw, cw)]
            w = jnp.maximum(q1, b_s[H + h:H + h + 1, pl.ds(j * cw, cw)]) * adjw
            nd = nd + jnp.dot(w, whn_s[pl.ds(j * cw, cw),
                                       h * 128:(h + 1) * 128],
                              preferred_element_type=jnp.float32)
        nds.append(nd)
    outs = []
    for h in range(H):
        o = nds[h][:, 0:HID] / nds[h][:, HID:HID + 1]
        outs.append(jnp.where(o > 0, o, jnp.exp(o) - 1.0))
    return jnp.concatenate(outs, axis=1)


def _body(x_ref, x0_ref, x1_ref, wua0_ref, wna0_ref, wnc0_ref,
          wua1_ref, wna1_ref, wnc1_ref, adj0l_ref, adj0r_ref,
          adj1l_ref, adj1r_ref,
          xb_ref, vxo_ref, wrwl_ref, blp_ref, out_ref,
          whn0_s, a0_s, b0_s, whn1_s, a1_s, b1_s):
    i = pl.program_id(0)

    @pl.when(i == 0)
    def _():
        _prologue_compute(x_ref[...], x0_ref[...], wua0_ref[...],
                          wna0_ref[...], wnc0_ref[...], whn0_s, a0_s, b0_s)
        _prologue_compute(x_ref[...], x1_ref[...], wua1_ref[...],
                          wna1_ref[...], wnc1_ref[...], whn1_s, a1_s, b1_s)

    o0 = _att_block([adj0l_ref, adj0r_ref], whn0_s, a0_s, b0_s, i)
    o1 = _att_block([adj1l_ref, adj1r_ref], whn1_s, a1_s, b1_s, i)

    sx = jnp.dot(xb_ref[...], vxo_ref[...],
                 preferred_element_type=jnp.float32)[:, 0:1]
    t0 = jnp.dot(o0, vxo_ref[...],
                 preferred_element_type=jnp.float32)[:, 1:2]
    t1 = jnp.dot(o1, vxo_ref[...],
                 preferred_element_type=jnp.float32)[:, 1:2]
    z0 = sx + t0
    z1 = sx + t1
    e0 = jnp.where(z0 >= 0, z0, ALPHA * z0)
    e1 = jnp.where(z1 >= 0, z1, ALPHA * z1)
    m = jnp.maximum(e0, e1)
    w0 = jnp.exp(e0 - m)
    w1 = jnp.exp(e1 - m)
    inv = 1.0 / (w0 + w1)
    mix = (w0 * inv) * o0 + (w1 * inv) * o1
    out_ref[...] = jnp.dot(mix, wrwl_ref[...],
                           preferred_element_type=jnp.float32) + blp_ref[0:1, :]


def kernel(x, x0, x1, adj0, adj1, Wu, Wn, au, an, W, Wr, ar, Wl, bl):
    rhid = Wr.shape[1]
    nclass = Wl.shape[1]
    wua = jnp.einsum('rhdk,rhk->rdh', Wu, au)
    wna = jnp.einsum('rhdk,rhk->rdh', Wn, an)
    wua_p = jnp.concatenate(
        [wua, jnp.zeros((R, DIMF, 8 - H), jnp.float32)], axis=2)
    wna_p = jnp.concatenate(
        [wna, jnp.zeros((R, DIMF, 8 - H), jnp.float32)], axis=2)
    wncat = jnp.concatenate([Wn[:, h] for h in range(H)], axis=2)

    v_x = W @ (Wr @ ar[:rhid])
    v_o = Wr @ ar[rhid:]
    vxo = jnp.zeros((DIMF, 8), jnp.float32)
    vxo = vxo.at[:, 0].set(v_x).at[:, 1].set(v_o)
    wrwl = Wr @ Wl
    blp = jnp.zeros((8, nclass), jnp.float32).at[0].set(bl)

    full = pl.BlockSpec((N, DIMF), lambda i: (0, 0))
    w8 = pl.BlockSpec((DIMF, 8), lambda i: (0, 0))
    wc = pl.BlockSpec((DIMF, H * HID), lambda i: (0, 0))
    adjl = pl.BlockSpec((BM, N // 2), lambda i: (i, 0))
    adjr = pl.BlockSpec((BM, N // 2), lambda i: (i, 1))
    scratch = [
        pltpu.VMEM((N, H * 128), jnp.bfloat16),
        pltpu.VMEM((N, 8), jnp.bfloat16),
        pltpu.VMEM((8, N), jnp.bfloat16),
    ] * R

    return pl.pallas_call(
        _body,
        grid=(N // BM,),
        in_specs=[full, full, full, w8, w8, wc, w8, w8, wc,
                  adjl, adjr, adjl, adjr,
                  pl.BlockSpec((BM, DIMF), lambda i: (i, 0)),
                  w8,
                  pl.BlockSpec((H * HID, 8), lambda i: (0, 0)),
                  pl.BlockSpec((8, 8), lambda i: (0, 0))],
        out_specs=pl.BlockSpec((BM, 8), lambda i: (i, 0)),
        out_shape=jax.ShapeDtypeStruct((N, 8), jnp.float32),
        scratch_shapes=scratch,
        compiler_params=pltpu.CompilerParams(
            dimension_semantics=("arbitrary",)),
    )(x, x0, x1, wua_p[0], wna_p[0], wncat[0], wua_p[1], wna_p[1], wncat[1],
      adj0, adj0, adj1, adj1, x, vxo, wrwl, blp)


# speedup vs baseline: 1.0400x; 1.0400x over previous
import jax
import jax.numpy as jnp
from jax.experimental import pallas as pl
from jax.experimental.pallas import tpu as pltpu

R = 2
H = 2
N = 4096
DIMF = 128
HID = 64
ALPHA = 0.2

BM = 512
NB = N // BM


def _prologue_compute(x, xr, wua, wna, wnc, whn_s, a_s, b_s):
    ones = jnp.ones((N, 1), jnp.float32)
    zer = jnp.zeros((N, 63), jnp.float32)
    whn = jnp.dot(xr, wnc, preferred_element_type=jnp.float32)
    whn_s[...] = jnp.concatenate(
        [whn[:, 0:HID], ones, zer, whn[:, HID:2 * HID], ones, zer],
        axis=1).astype(jnp.bfloat16)
    su = jnp.dot(x, wua, preferred_element_type=jnp.float32)
    a_s[...] = jnp.concatenate(
        [jnp.exp((1.0 - ALPHA) * su[:, 0:H]),
         jnp.zeros((N, 8 - H), jnp.float32)], axis=1).astype(jnp.bfloat16)
    sn = jnp.dot(xr, wna, preferred_element_type=jnp.float32)
    b_s[...] = jnp.concatenate(
        [jnp.exp(sn[:, 0:H]), jnp.exp(ALPHA * sn[:, 0:H]),
         jnp.zeros((N, 8 - 2 * H), jnp.float32)],
        axis=1).T.astype(jnp.bfloat16)


def _att_block(adj_refs, whn_s, a_s, b_s, i):
    nh = len(adj_refs)
    cw = N // nh
    a = a_s[pl.ds(i * BM, BM), :]
    nds = []
    for h in range(H):
        nd = jnp.zeros((BM, 128), jnp.float32)
        for j, adj_ref in enumerate(adj_refs):
            adjw = adj_ref[...].astype(jnp.bfloat16)
            q1 = a[:, h:h + 1] * b_s[h:h + 1, pl.ds(j * cw, cw)]
            w = jnp.maximum(q1, b_s[H + h:H + h + 1, pl.ds(j * cw, cw)]) * adjw
            nd = nd + jnp.dot(w, whn_s[pl.ds(j * cw, cw),
                                       h * 128:(h + 1) * 128],
                              preferred_element_type=jnp.float32)
        nds.append(nd)
    outs = []
    for h in range(H):
        o = nds[h][:, 0:HID] / nds[h][:, HID:HID + 1]
        outs.append(jnp.where(o > 0, o, jnp.exp(o) - 1.0))
    return jnp.concatenate(outs, axis=1)


def _body(x_ref, x0_ref, x1_ref, wua0_ref, wna0_ref, wnc0_ref,
          wua1_ref, wna1_ref, wnc1_ref, adj0l_ref, adj0r_ref,
          adj1l_ref, adj1r_ref,
          xb_ref, vxo_ref, wrwl_ref, blp_ref, out_ref,
          whn0_s, a0_s, b0_s, whn1_s, a1_s, b1_s, o0_s):
    i = pl.program_id(0)

    @pl.when(i == 0)
    def _():
        _prologue_compute(x_ref[...], x0_ref[...], wua0_ref[...],
                          wna0_ref[...], wnc0_ref[...], whn0_s, a0_s, b0_s)

    @pl.when(i == 1)
    def _():
        _prologue_compute(x_ref[...], x1_ref[...], wua1_ref[...],
                          wna1_ref[...], wnc1_ref[...], whn1_s, a1_s, b1_s)

    @pl.when(i > 0)
    def _():
        o1 = _att_block([adj1l_ref, adj1r_ref], whn1_s, a1_s, b1_s, i - 1)
        o0 = o0_s[...]
        sx = jnp.dot(xb_ref[...], vxo_ref[...],
                     preferred_element_type=jnp.float32)[:, 0:1]
        t0 = jnp.dot(o0, vxo_ref[...],
                     preferred_element_type=jnp.float32)[:, 1:2]
        t1 = jnp.dot(o1, vxo_ref[...],
                     preferred_element_type=jnp.float32)[:, 1:2]
        z0 = sx + t0
        z1 = sx + t1
        e0 = jnp.where(z0 >= 0, z0, ALPHA * z0)
        e1 = jnp.where(z1 >= 0, z1, ALPHA * z1)
        m = jnp.maximum(e0, e1)
        w0 = jnp.exp(e0 - m)
        w1 = jnp.exp(e1 - m)
        inv = 1.0 / (w0 + w1)
        mix = (w0 * inv) * o0 + (w1 * inv) * o1
        out_ref[...] = jnp.dot(
            mix, wrwl_ref[...],
            preferred_element_type=jnp.float32) + blp_ref[0:1, :]

    @pl.when(i < NB)
    def _():
        o0_s[...] = _att_block([adj0l_ref, adj0r_ref], whn0_s, a0_s, b0_s, i)


def kernel(x, x0, x1, adj0, adj1, Wu, Wn, au, an, W, Wr, ar, Wl, bl):
    rhid = Wr.shape[1]
    nclass = Wl.shape[1]
    wua = jnp.einsum('rhdk,rhk->rdh', Wu, au)
    wna = jnp.einsum('rhdk,rhk->rdh', Wn, an)
    wua_p = jnp.concatenate(
        [wua, jnp.zeros((R, DIMF, 8 - H), jnp.float32)], axis=2)
    wna_p = jnp.concatenate(
        [wna, jnp.zeros((R, DIMF, 8 - H), jnp.float32)], axis=2)
    wncat = jnp.concatenate([Wn[:, h] for h in range(H)], axis=2)

    v_x = W @ (Wr @ ar[:rhid])
    v_o = Wr @ ar[rhid:]
    vxo = jnp.zeros((DIMF, 8), jnp.float32)
    vxo = vxo.at[:, 0].set(v_x).at[:, 1].set(v_o)
    wrwl = Wr @ Wl
    blp = jnp.zeros((8, nclass), jnp.float32).at[0].set(bl)

    full = pl.BlockSpec((N, DIMF), lambda i: (0, 0))
    w8 = pl.BlockSpec((DIMF, 8), lambda i: (0, 0))
    wc = pl.BlockSpec((DIMF, H * HID), lambda i: (0, 0))
    adj0l = pl.BlockSpec((BM, N // 2), lambda i: (jnp.minimum(i, NB - 1), 0))
    adj0r = pl.BlockSpec((BM, N // 2), lambda i: (jnp.minimum(i, NB - 1), 1))
    adj1l = pl.BlockSpec((BM, N // 2), lambda i: (jnp.maximum(i - 1, 0), 0))
    adj1r = pl.BlockSpec((BM, N // 2), lambda i: (jnp.maximum(i - 1, 0), 1))
    prev = lambda i: (jnp.maximum(i - 1, 0), 0)
    scratch = [
        pltpu.VMEM((N, H * 128), jnp.bfloat16),
        pltpu.VMEM((N, 8), jnp.bfloat16),
        pltpu.VMEM((8, N), jnp.bfloat16),
    ] * R + [
        pltpu.VMEM((BM, H * HID), jnp.float32),
    ]

    return pl.pallas_call(
        _body,
        grid=(NB + 1,),
        in_specs=[full, full, full, w8, w8, wc, w8, w8, wc,
                  adj0l, adj0r, adj1l, adj1r,
                  pl.BlockSpec((BM, DIMF), prev),
                  w8,
                  pl.BlockSpec((H * HID, 8), lambda i: (0, 0)),
                  pl.BlockSpec((8, 8), lambda i: (0, 0))],
        out_specs=pl.BlockSpec((BM, 8), prev),
        out_shape=jax.ShapeDtypeStruct((N, 8), jnp.float32),
        scratch_shapes=scratch,
        compiler_params=pltpu.CompilerParams(
            dimension_semantics=("arbitrary",)),
    )(x, x0, x1, wua_p[0], wna_p[0], wncat[0], wua_p[1], wna_p[1], wncat[1],
      adj0, adj0, adj1, adj1, x, vxo, wrwl, blp)


# speedup vs baseline: 1.0669x; 1.0259x over previous
import jax
import jax.numpy as jnp
from jax.experimental import pallas as pl
from jax.experimental.pallas import tpu as pltpu

R = 2
H = 2
N = 4096
DIMF = 128
HID = 64
ALPHA = 0.2

BM = 512
NB = N // BM


def _prologue_compute(x, xr, wua, wna, wnc, whn_s, a_s, b_s):
    ones = jnp.ones((N, 1), jnp.float32)
    zer = jnp.zeros((N, 63), jnp.float32)
    whn = jnp.dot(xr, wnc, preferred_element_type=jnp.float32)
    whn_s[...] = jnp.concatenate(
        [whn[:, 0:HID], ones, zer, whn[:, HID:2 * HID], ones, zer],
        axis=1).astype(jnp.bfloat16)
    su = jnp.dot(x, wua, preferred_element_type=jnp.float32)
    a_s[...] = jnp.concatenate(
        [jnp.exp((1.0 - ALPHA) * su[:, 0:H]),
         jnp.zeros((N, 8 - H), jnp.float32)], axis=1).astype(jnp.bfloat16)
    sn = jnp.dot(xr, wna, preferred_element_type=jnp.float32)
    b_s[...] = jnp.concatenate(
        [jnp.exp(sn[:, 0:H]), jnp.exp(ALPHA * sn[:, 0:H]),
         jnp.zeros((N, 8 - 2 * H), jnp.float32)],
        axis=1).T.astype(jnp.bfloat16)


def _att_block(adj_refs, whn_s, a_s, b_s, i):
    nh = len(adj_refs)
    cw = N // nh
    a = a_s[pl.ds(i * BM, BM), :]
    nds = []
    for h in range(H):
        nd = jnp.zeros((BM, 128), jnp.float32)
        for j, adj_ref in enumerate(adj_refs):
            adjw = adj_ref[...].astype(jnp.bfloat16)
            q1 = a[:, h:h + 1] * b_s[h:h + 1, pl.ds(j * cw, cw)]
            w = jnp.maximum(q1, b_s[H + h:H + h + 1, pl.ds(j * cw, cw)]) * adjw
            nd = nd + jnp.dot(w, whn_s[pl.ds(j * cw, cw),
                                       h * 128:(h + 1) * 128],
                              preferred_element_type=jnp.float32)
        nds.append(nd)
    outs = []
    for h in range(H):
        o = nds[h][:, 0:HID] / nds[h][:, HID:HID + 1]
        outs.append(jnp.where(o > 0, o, jnp.exp(o) - 1.0))
    return jnp.concatenate(outs, axis=1)


def _body(x_ref, x0_ref, x1_ref, wua0_ref, wna0_ref, wnc0_ref,
          wua1_ref, wna1_ref, wnc1_ref, adj0l_ref, adj0r_ref,
          adj1l_ref, adj1r_ref,
          xb_ref, vxo_ref, wrwl_ref, blp_ref, out_ref,
          whn0_s, a0_s, b0_s, whn1_s, a1_s, b1_s):
    i = pl.program_id(0)

    @pl.when(i == 0)
    def _():
        _prologue_compute(x_ref[...], x0_ref[...], wua0_ref[...],
                          wna0_ref[...], wnc0_ref[...], whn0_s, a0_s, b0_s)
        _prologue_compute(x_ref[...], x1_ref[...], wua1_ref[...],
                          wna1_ref[...], wnc1_ref[...], whn1_s, a1_s, b1_s)

    o0 = _att_block([adj0l_ref, adj0r_ref], whn0_s, a0_s, b0_s, i)
    o1 = _att_block([adj1l_ref, adj1r_ref], whn1_s, a1_s, b1_s, i)

    sx = jnp.dot(xb_ref[...], vxo_ref[...],
                 preferred_element_type=jnp.float32)[:, 0:1]
    t0 = jnp.dot(o0, vxo_ref[...],
                 preferred_element_type=jnp.float32)[:, 1:2]
    t1 = jnp.dot(o1, vxo_ref[...],
                 preferred_element_type=jnp.float32)[:, 1:2]
    z0 = sx + t0
    z1 = sx + t1
    e0 = jnp.where(z0 >= 0, z0, ALPHA * z0)
    e1 = jnp.where(z1 >= 0, z1, ALPHA * z1)
    m = jnp.maximum(e0, e1)
    w0 = jnp.exp(e0 - m)
    w1 = jnp.exp(e1 - m)
    inv = 1.0 / (w0 + w1)
    mix = (w0 * inv) * o0 + (w1 * inv) * o1
    out_ref[...] = jnp.dot(mix, wrwl_ref[...],
                           preferred_element_type=jnp.float32) + blp_ref[0:1, :]


def kernel(x, x0, x1, adj0, adj1, Wu, Wn, au, an, W, Wr, ar, Wl, bl):
    rhid = Wr.shape[1]
    nclass = Wl.shape[1]
    wua = jnp.einsum('rhdk,rhk->rdh', Wu, au)
    wna = jnp.einsum('rhdk,rhk->rdh', Wn, an)
    wua_p = jnp.concatenate(
        [wua, jnp.zeros((R, DIMF, 8 - H), jnp.float32)], axis=2)
    wna_p = jnp.concatenate(
        [wna, jnp.zeros((R, DIMF, 8 - H), jnp.float32)], axis=2)
    wncat = jnp.concatenate([Wn[:, h] for h in range(H)], axis=2)

    v_x = W @ (Wr @ ar[:rhid])
    v_o = Wr @ ar[rhid:]
    vxo = jnp.zeros((DIMF, 8), jnp.float32)
    vxo = vxo.at[:, 0].set(v_x).at[:, 1].set(v_o)
    wrwl = Wr @ Wl
    blp = jnp.zeros((8, nclass), jnp.float32).at[0].set(bl)

    full = pl.BlockSpec((N, DIMF), lambda i: (0, 0))
    w8 = pl.BlockSpec((DIMF, 8), lambda i: (0, 0))
    wc = pl.BlockSpec((DIMF, H * HID), lambda i: (0, 0))
    adjl = pl.BlockSpec((BM, N // 2), lambda i: (i, 0))
    adjr = pl.BlockSpec((BM, N // 2), lambda i: (i, 1))
    scratch = [
        pltpu.VMEM((N, H * 128), jnp.bfloat16),
        pltpu.VMEM((N, 8), jnp.bfloat16),
        pltpu.VMEM((8, N), jnp.bfloat16),
    ] * R

    return pl.pallas_call(
        _body,
        grid=(NB,),
        in_specs=[full, full, full, w8, w8, wc, w8, w8, wc,
                  adjl, adjr, adjl, adjr,
                  pl.BlockSpec((BM, DIMF), lambda i: (i, 0)),
                  w8,
                  pl.BlockSpec((H * HID, 8), lambda i: (0, 0)),
                  pl.BlockSpec((8, 8), lambda i: (0, 0))],
        out_specs=pl.BlockSpec((BM, 8), lambda i: (i, 0)),
        out_shape=jax.ShapeDtypeStruct((N, 8), jnp.float32),
        scratch_shapes=scratch,
        compiler_params=pltpu.CompilerParams(
            dimension_semantics=("arbitrary",)),
    )(x, x0, x1, wua_p[0], wna_p[0], wncat[0], wua_p[1], wna_p[1], wncat[1],
      adj0, adj0, adj1, adj1, x, vxo, wrwl, blp)
